# baseline (speedup 1.0000x reference)
import functools

import jax
import jax.numpy as jnp
from jax import lax
from jax.experimental import pallas as pl
from jax.experimental.pallas import tpu as pltpu
from jax.experimental.pallas import tpu_sc as plsc

N = 1_000_000
GRID = 512
TAB = GRID * GRID
NG = 2
NP = N // NG
C = 4_000
NCHUNK = NP // C
NW = 32
VPC = C // 16


def _mesh_kernel(x1, y1, x2, y2, ii, jj, t0, t1):
    mesh = plsc.VectorSubcoreMesh(core_axis_name="c", subcore_axis_name="s",
                                  num_cores=2, num_subcores=16)

    @functools.partial(
        pl.kernel,
        mesh=mesh,
        out_type=jax.ShapeDtypeStruct((NW, 16), jnp.float32),
        scratch_types=[
            pltpu.VMEM_SHARED((TAB,), jnp.float32),
            pltpu.VMEM_SHARED((TAB,), jnp.float32),
            pltpu.VMEM((C,), jnp.float32),
            pltpu.VMEM((C,), jnp.float32),
            pltpu.VMEM((C,), jnp.float32),
            pltpu.VMEM((C,), jnp.float32),
            pltpu.VMEM((C,), jnp.int32),
            pltpu.VMEM((C,), jnp.int32),
            pltpu.VMEM((C,), jnp.int32),
            pltpu.VMEM((C,), jnp.float32),
            pltpu.VMEM((C,), jnp.float32),
            pltpu.VMEM((16,), jnp.float32),
            pltpu.SemaphoreType.DMA,
        ],
    )
    def body(x1_hbm, y1_hbm, x2_hbm, y2_hbm, ii_hbm, jj_hbm, t0_hbm, t1_hbm,
             out_hbm, t0_sh, t1_sh, x1_v, y1_v, x2_v, y2_v, ii_v, jj_v,
             flat_v, q0_v, q1_v, acc_v, sem):
        cid = lax.axis_index("c")
        sid = lax.axis_index("s")
        wid = sid * 2 + cid

        seg = TAB // 16
        pltpu.sync_copy(t0_hbm.at[pl.ds(sid * seg, seg)],
                        t0_sh.at[pl.ds(sid * seg, seg)])
        pltpu.sync_copy(t1_hbm.at[pl.ds(sid * seg, seg)],
                        t1_sh.at[pl.ds(sid * seg, seg)])
        plsc.subcore_barrier()

        def chunk_body(t, acc):
            base = (wid + NW * t) * C
            pltpu.sync_copy(ii_hbm.at[pl.ds(base, C)], ii_v)
            pltpu.sync_copy(jj_hbm.at[pl.ds(base, C)], jj_v)
            pltpu.sync_copy(x2_hbm.at[pl.ds(base, C)], x2_v)
            pltpu.sync_copy(y2_hbm.at[pl.ds(base, C)], y2_v)
            pltpu.sync_copy(x1_hbm.at[pl.ds(base, C)], x1_v)
            pltpu.sync_copy(y1_hbm.at[pl.ds(base, C)], y1_v)

            def flat_body(k, _):
                i = ii_v[pl.ds(k * 16, 16)]
                j = jj_v[pl.ds(k * 16, 16)]
                flat_v[pl.ds(k * 16, 16)] = i * GRID + j
                return 0

            lax.fori_loop(0, VPC, flat_body, 0, unroll=8)

            pltpu.async_copy(t0_sh.at[flat_v], q0_v, sem).wait()
            pltpu.async_copy(t1_sh.at[flat_v], q1_v, sem).wait()

            def comp_body(k, a):
                sl = pl.ds(k * 16, 16)
                q0 = q0_v[sl]
                q1 = q1_v[sl]
                rx = x2_v[sl] - q0
                ry = y2_v[sl] - q1
                s = (rx * rx) * (ry * ry)
                d0 = x1_v[sl] - s * q0
                d1 = y1_v[sl] - s * q1
                return a + d0 * d0 + d1 * d1

            return lax.fori_loop(0, VPC, comp_body, acc, unroll=8)

        nt = (NCHUNK - wid + NW - 1) // NW
        acc = lax.fori_loop(0, nt, chunk_body, jnp.zeros((16,), jnp.float32))
        acc_v[...] = acc
        pltpu.sync_copy(acc_v, out_hbm.at[wid])

    return body(x1, y1, x2, y2, ii, jj, t0, t1)


def kernel(ch1, ch2, CP_locs, CP_idx):
    t0 = CP_locs[:, :, 0].reshape(-1)
    t1 = CP_locs[:, :, 1].reshape(-1)
    total = jnp.float32(0.0)
    for g in range(NG):
        sl = slice(g * NP, (g + 1) * NP)
        partials = _mesh_kernel(
            ch1[sl, 0], ch1[sl, 1],
            ch2[sl, 0], ch2[sl, 1],
            CP_idx[sl, 0], CP_idx[sl, 1],
            t0, t1,
        )
        total = total + jnp.sum(partials)
    return total * jnp.float32(0.5 / N)


# speedup vs baseline: 1.1832x; 1.1832x over previous
import functools

import jax
import jax.numpy as jnp
from jax import lax
from jax.experimental import pallas as pl
from jax.experimental.pallas import tpu as pltpu
from jax.experimental.pallas import tpu_sc as plsc

N = 1_000_000
GRID = 512
TAB = GRID * GRID
C = 4_000
NCHUNK = N // C
NW = 32
VPC = C // 16
MIN_NT = NCHUNK // NW
MAX_NT = -(-NCHUNK // NW)


def _mesh_kernel(x1, y1, x2, y2, ii, jj, t0, t1):
    mesh = plsc.VectorSubcoreMesh(core_axis_name="c", subcore_axis_name="s",
                                  num_cores=2, num_subcores=16)

    fbuf = lambda: pltpu.VMEM((C,), jnp.float32)
    ibuf = lambda: pltpu.VMEM((C,), jnp.int32)

    @functools.partial(
        pl.kernel,
        mesh=mesh,
        out_type=jax.ShapeDtypeStruct((NW, 16), jnp.float32),
        scratch_types=[
            pltpu.VMEM_SHARED((TAB,), jnp.float32),
            pltpu.VMEM_SHARED((TAB,), jnp.float32),
            fbuf(), fbuf(), fbuf(), fbuf(), ibuf(), ibuf(), ibuf(), fbuf(),
            fbuf(),
            fbuf(), fbuf(), fbuf(), fbuf(), ibuf(), ibuf(), ibuf(), fbuf(),
            fbuf(),
            pltpu.VMEM((16,), jnp.float32),
            pltpu.SemaphoreType.DMA,
            pltpu.SemaphoreType.DMA,
        ],
    )
    def body(x1_hbm, y1_hbm, x2_hbm, y2_hbm, ii_hbm, jj_hbm, t0_hbm, t1_hbm,
             out_hbm, t0_sh, t1_sh,
             x1a, y1a, x2a, y2a, iia, jja, fla, q0a, q1a,
             x1b, y1b, x2b, y2b, iib, jjb, flb, q0b, q1b,
             acc_v, sem_in, sem_g):
        cid = lax.axis_index("c")
        sid = lax.axis_index("s")
        wid = sid * 2 + cid

        seg = TAB // 16
        pltpu.sync_copy(t0_hbm.at[pl.ds(sid * seg, seg)],
                        t0_sh.at[pl.ds(sid * seg, seg)])
        pltpu.sync_copy(t1_hbm.at[pl.ds(sid * seg, seg)],
                        t1_sh.at[pl.ds(sid * seg, seg)])
        plsc.subcore_barrier()

        nt = (NCHUNK - wid + NW - 1) // NW
        hbm = (x1_hbm, y1_hbm, x2_hbm, y2_hbm, ii_hbm, jj_hbm)
        bufs = ((x1a, y1a, x2a, y2a, iia, jja, fla, q0a, q1a),
                (x1b, y1b, x2b, y2b, iib, jjb, flb, q0b, q1b))

        def issue_inputs(t, b):
            base = (wid + NW * t) * C
            for h, v in zip(hbm, bufs[b][:6]):
                pltpu.async_copy(h.at[pl.ds(base, C)], v, sem_in)

        def stage(t, b):
            x1v, y1v, x2v, y2v, iiv, jjv, flv, q0v, q1v = bufs[b]
            for h, v in zip(hbm, bufs[b][:6]):
                pltpu.make_async_copy(h.at[pl.ds(0, C)], v, sem_in).wait()

            def flat_body(k, _):
                sl = pl.ds(k * 16, 16)
                flv[sl] = iiv[sl] * GRID + jjv[sl]
                return 0

            lax.fori_loop(0, VPC, flat_body, 0, unroll=8)
            pltpu.async_copy(t0_sh.at[flv], q0v, sem_g)
            pltpu.async_copy(t1_sh.at[flv], q1v, sem_g)

        def finish(b):
            x1v, y1v, x2v, y2v, iiv, jjv, flv, q0v, q1v = bufs[b]
            pltpu.make_async_copy(t0_hbm.at[pl.ds(0, C)], q0v, sem_g).wait()
            pltpu.make_async_copy(t1_hbm.at[pl.ds(0, C)], q1v, sem_g).wait()

            def comp_body(k, a):
                sl = pl.ds(k * 16, 16)
                q0 = q0v[sl]
                q1 = q1v[sl]
                rx = x2v[sl] - q0
                ry = y2v[sl] - q1
                s = (rx * rx) * (ry * ry)
                d0 = x1v[sl] - s * q0
                d1 = y1v[sl] - s * q1
                return a + d0 * d0 + d1 * d1

            acc_v[...] = lax.fori_loop(0, VPC, comp_body, acc_v[...],
                                       unroll=8)

        acc_v[...] = jnp.zeros((16,), jnp.float32)
        issue_inputs(0, 0)
        stage(0, 0)
        issue_inputs(1, 1)
        for t in range(1, MAX_NT):
            b = t % 2
            if t < MIN_NT:
                stage(t, b)
            else:
                @pl.when(t < nt)
                def _(t=t, b=b):
                    stage(t, b)
            finish(1 - b)
            if t + 1 < MAX_NT:
                @pl.when(t + 1 < nt)
                def _(t=t, b=b):
                    issue_inputs(t + 1, 1 - b)

        @pl.when(nt == MAX_NT)
        def _():
            finish((MAX_NT - 1) % 2)

        pltpu.sync_copy(acc_v, out_hbm.at[wid])

    return body(x1, y1, x2, y2, ii, jj, t0, t1)


def kernel(ch1, ch2, CP_locs, CP_idx):
    partials = _mesh_kernel(
        ch1[:, 0], ch1[:, 1],
        ch2[:, 0], ch2[:, 1],
        CP_idx[:, 0], CP_idx[:, 1],
        CP_locs[:, :, 0].reshape(-1), CP_locs[:, :, 1].reshape(-1),
    )
    return jnp.sum(partials) * jnp.float32(0.5 / N)
